# Initial kernel scaffold; baseline (speedup 1.0000x reference)
import jax
import jax.numpy as jnp
from jax.experimental import pallas as pl


def kernel(pred_logits, pred_boxes, pred_actions, tgt_labels, tgt_boxes, tgt_actions):
    raise NotImplementedError("write your pallas kernel here")


# speedup vs baseline: 18.9389x; 18.9389x over previous
import jax
import jax.numpy as jnp
from jax import lax
from jax.experimental import pallas as pl

_B, _Q, _T = 4, 10000, 50
_NC1 = 81
_NA = 117
_CH = 2000
_GSZ = 200
_G = _Q // _GSZ
_BIGF = 1e9
_BIGI = 1 << 30


def _matcher_kernel(logits_ref, boxes_ref, actions_ref, tlab_ref, tboxT_ref,
                    tactT_ref, c_ref, rows_ref, cols_ref):
    f32 = jnp.float32

    tlab = tlab_ref[0]
    tboxT = tboxT_ref[0]
    tactT = tactT_ref[0]

    tcx, tcy, tw, th = (tboxT[0:1, :], tboxT[1:2, :], tboxT[2:3, :], tboxT[3:4, :])
    tx0 = tcx - 0.5 * tw
    ty0 = tcy - 0.5 * th
    tx1 = tcx + 0.5 * tw
    ty1 = tcy + 0.5 * th
    tarea = (tx1 - tx0) * (ty1 - ty0)

    onehot = (lax.broadcasted_iota(jnp.int32, (_NC1, _T), 0) == tlab).astype(f32)
    one_minus_tact = 1.0 - tactT

    def stage1_body(k, carry):
        sl = pl.ds(k * _CH, _CH)
        logits = logits_ref[0, sl, :]
        m = jnp.max(logits, axis=-1, keepdims=True)
        e = jnp.exp(logits - m)
        s = jnp.sum(e, axis=-1, keepdims=True)
        prob = e / s
        cost_class = -jnp.dot(prob, onehot, preferred_element_type=f32)

        boxes = boxes_ref[0, sl, :]
        qcx, qcy, qw, qh = (boxes[:, 0:1], boxes[:, 1:2], boxes[:, 2:3], boxes[:, 3:4])
        cost_bbox = (jnp.abs(qcx - tcx) + jnp.abs(qcy - tcy)
                     + jnp.abs(qw - tw) + jnp.abs(qh - th))

        qx0 = qcx - 0.5 * qw
        qy0 = qcy - 0.5 * qh
        qx1 = qcx + 0.5 * qw
        qy1 = qcy + 0.5 * qh
        qarea = (qx1 - qx0) * (qy1 - qy0)
        wx = jnp.maximum(jnp.minimum(qx1, tx1) - jnp.maximum(qx0, tx0), 0.0)
        wy = jnp.maximum(jnp.minimum(qy1, ty1) - jnp.maximum(qy0, ty0), 0.0)
        inter = wx * wy
        union = qarea + tarea - inter
        iou = inter / (union + 1e-8)
        ew = jnp.maximum(jnp.maximum(qx1, tx1) - jnp.minimum(qx0, tx0), 0.0)
        eh = jnp.maximum(jnp.maximum(qy1, ty1) - jnp.minimum(qy0, ty0), 0.0)
        enclose = ew * eh
        cost_giou = -(iou - (enclose - union) / (enclose + 1e-8))

        acts = actions_ref[0, sl, :]
        p = jax.nn.sigmoid(acts)
        lp = jnp.log(p + 1e-8)
        lq = jnp.log(1.0 - p + 1e-8)
        cost_action = -(jnp.dot(lp, tactT, preferred_element_type=f32)
                        + jnp.dot(lq, one_minus_tact, preferred_element_type=f32)) / _NA

        c_ref[0, sl, :] = (5.0 * cost_bbox + 1.0 * cost_class
                           + 2.0 * cost_giou + 1.0 * cost_action)
        return carry

    lax.fori_loop(0, _Q // _CH, stage1_body, 0)

    iota_gsz_t = lax.broadcasted_iota(jnp.int32, (_GSZ, _T), 0)
    iota_g = lax.broadcasted_iota(jnp.int32, (_G, _T), 0)
    iota_t = lax.broadcasted_iota(jnp.int32, (_G, _T), 1)

    def minima_body(gr, carry):
        gmin, garg = carry
        sub = c_ref[0, pl.ds(gr * _GSZ, _GSZ), :]
        mv = jnp.min(sub, axis=0, keepdims=True)
        rid = jnp.where(sub == mv, iota_gsz_t + gr * _GSZ, _BIGI)
        ra = jnp.min(rid, axis=0, keepdims=True)
        rowsel = iota_g == gr
        return jnp.where(rowsel, mv, gmin), jnp.where(rowsel, ra, garg)

    gmin0, garg0 = lax.fori_loop(
        0, _G, minima_body,
        (jnp.zeros((_G, _T), f32), jnp.zeros((_G, _T), jnp.int32)))

    flat = iota_g * _T + iota_t
    lane = lax.broadcasted_iota(jnp.int32, (1, _T), 1)
    iota_rm_rl = lax.broadcasted_iota(jnp.int32, (_GSZ, _G), 0)
    iota_rm_gr = lax.broadcasted_iota(jnp.int32, (_GSZ, _G), 1)

    def body(i, state):
        gmin, garg, colmask, rowmask, rows_v, cols_v = state
        masked = gmin + colmask
        mv = jnp.min(masked)
        fidx = jnp.min(jnp.where(masked == mv, flat, _BIGI))
        gr = fidx // _T
        c = fidx - gr * _T
        r = jnp.min(jnp.where(flat == fidx, garg, _BIGI))
        rows_v = jnp.where(lane == i, r, rows_v)
        cols_v = jnp.where(lane == i, c, cols_v)
        colmask = colmask + jnp.where(lane == c, _BIGF, 0.0)
        rl = r - gr * _GSZ
        rowmask = rowmask + jnp.where((iota_rm_rl == rl) & (iota_rm_gr == gr),
                                      _BIGF, 0.0)
        sub = c_ref[0, pl.ds(gr * _GSZ, _GSZ), :]
        rmcol = jnp.min(jnp.where(iota_rm_gr == gr, rowmask, jnp.float32(1e30)),
                        axis=1, keepdims=True)
        subm = sub + rmcol
        nmin = jnp.min(subm, axis=0, keepdims=True)
        narg = jnp.min(jnp.where(subm == nmin, iota_gsz_t + gr * _GSZ, _BIGI),
                       axis=0, keepdims=True)
        rowsel = iota_g == gr
        gmin = jnp.where(rowsel, nmin, gmin)
        garg = jnp.where(rowsel, narg, garg)
        return gmin, garg, colmask, rowmask, rows_v, cols_v

    init = (gmin0, garg0,
            jnp.zeros((1, _T), f32), jnp.zeros((_GSZ, _G), f32),
            jnp.zeros((1, _T), jnp.int32), jnp.zeros((1, _T), jnp.int32))
    _, _, _, _, rows_v, cols_v = lax.fori_loop(0, _T, body, init)
    rows_ref[0] = rows_v
    cols_ref[0] = cols_v


def kernel(pred_logits, pred_boxes, pred_actions, tgt_labels, tgt_boxes, tgt_actions):
    B, Q, _ = pred_logits.shape
    T = tgt_labels.shape[1]
    tlab3 = tgt_labels.astype(jnp.int32).reshape(B, 1, T)
    tboxT = tgt_boxes.transpose(0, 2, 1)
    tactT = tgt_actions.transpose(0, 2, 1)

    c_out, rows3, cols3 = pl.pallas_call(
        _matcher_kernel,
        grid=(B,),
        in_specs=[
            pl.BlockSpec((1, Q, _NC1), lambda b: (b, 0, 0)),
            pl.BlockSpec((1, Q, 4), lambda b: (b, 0, 0)),
            pl.BlockSpec((1, Q, _NA), lambda b: (b, 0, 0)),
            pl.BlockSpec((1, 1, T), lambda b: (b, 0, 0)),
            pl.BlockSpec((1, 4, T), lambda b: (b, 0, 0)),
            pl.BlockSpec((1, _NA, T), lambda b: (b, 0, 0)),
        ],
        out_specs=[
            pl.BlockSpec((1, Q, T), lambda b: (b, 0, 0)),
            pl.BlockSpec((1, 1, T), lambda b: (b, 0, 0)),
            pl.BlockSpec((1, 1, T), lambda b: (b, 0, 0)),
        ],
        out_shape=[
            jax.ShapeDtypeStruct((B, Q, T), jnp.float32),
            jax.ShapeDtypeStruct((B, 1, T), jnp.int32),
            jax.ShapeDtypeStruct((B, 1, T), jnp.int32),
        ],
    )(pred_logits, pred_boxes, pred_actions, tlab3, tboxT, tactT)
    return c_out, rows3.reshape(B, T), cols3.reshape(B, T)
